# baseline (speedup 1.0000x reference)
import functools

import jax
import jax.numpy as jnp
from jax import lax
from jax.experimental import pallas as pl
from jax.experimental.pallas import tpu as pltpu
from jax.experimental.pallas import tpu_sc as plsc

N_EDGES = 160000
N_TRIP = 160000
H = 128

NC = 2
NS = 16
NW = NC * NS


_BA = 1600


def _silu(v):
    return v * (1.0 / (1.0 + jnp.exp(-v)))


def _pre_body(x_ref, rbf_ref, wji_ref, bji_ref, wkj_ref, bkj_ref, wrbf_ref,
              xji_ref, xkj_ref):
    xb = x_ref[...]
    xji_ref[...] = _silu(jnp.dot(xb, wji_ref[...],
                                 preferred_element_type=jnp.float32) + bji_ref[...])
    rbf_p = jnp.dot(rbf_ref[...], wrbf_ref[...], preferred_element_type=jnp.float32)
    xkj_ref[...] = _silu(jnp.dot(xb, wkj_ref[...],
                                 preferred_element_type=jnp.float32) + bkj_ref[...]) * rbf_p


def _pre_call(x, rbf, wjiT, bji, wkjT, bkj, wrbfT):
    n = x.shape[0]
    grid = (n // _BA,)
    row_spec = pl.BlockSpec((_BA, H), lambda i: (i, 0))
    full = lambda shape: pl.BlockSpec(shape, lambda i: tuple(0 for _ in shape))
    return pl.pallas_call(
        _pre_body,
        grid=grid,
        in_specs=[
            row_spec,
            pl.BlockSpec((_BA, 6), lambda i: (i, 0)),
            full((H, H)), full((1, H)), full((H, H)), full((1, H)), full((6, H)),
        ],
        out_specs=[row_spec, row_spec],
        out_shape=[jax.ShapeDtypeStruct((n, H), jnp.float32),
                   jax.ShapeDtypeStruct((n, H), jnp.float32)],
    )(x, rbf, wjiT, bji, wkjT, bkj, wrbfT)


_BC = 800


def _einsum_body(g_ref, sbf_ref, wsbf_ref, w2_ref, msg_ref):
    g = g_ref[...]
    sbfp = jnp.dot(sbf_ref[...], wsbf_ref[...],
                   preferred_element_type=jnp.float32)
    parts = [g * sbfp[:, j:j + 1] for j in range(8)]
    g2 = jnp.concatenate(parts, axis=1)
    msg_ref[...] = jnp.dot(g2, w2_ref[...], preferred_element_type=jnp.float32)


def _einsum_call(gathered, sbf, wsbfT, w2):
    n = gathered.shape[0]
    grid = (n // _BC,)
    row_spec = pl.BlockSpec((_BC, H), lambda i: (i, 0))
    full = lambda shape: pl.BlockSpec(shape, lambda i: tuple(0 for _ in shape))
    return pl.pallas_call(
        _einsum_body,
        grid=grid,
        in_specs=[
            row_spec,
            pl.BlockSpec((_BC, 42), lambda i: (i, 0)),
            full((42, 8)), full((8 * H, H)),
        ],
        out_specs=row_spec,
        out_shape=jax.ShapeDtypeStruct((n, H), jnp.float32),
    )(gathered, sbf, wsbfT, w2)


def _post_body(xji_ref, agg_ref, x_ref,
               bw1_ref, bb1_ref, bw2_ref, bb2_ref,
               lw_ref, lb_ref,
               aw1a_ref, ab1a_ref, aw2a_ref, ab2a_ref,
               aw1b_ref, ab1b_ref, aw2b_ref, ab2b_ref,
               out_ref):
    dot = lambda a, w: jnp.dot(a, w[...], preferred_element_type=jnp.float32)
    h = xji_ref[...] + agg_ref[...]
    h = h + _silu(dot(_silu(dot(h, bw1_ref) + bb1_ref[...]), bw2_ref) + bb2_ref[...])
    h = _silu(dot(h, lw_ref) + lb_ref[...]) + x_ref[...]
    h = h + _silu(dot(_silu(dot(h, aw1a_ref) + ab1a_ref[...]), aw2a_ref) + ab2a_ref[...])
    h = h + _silu(dot(_silu(dot(h, aw1b_ref) + ab1b_ref[...]), aw2b_ref) + ab2b_ref[...])
    out_ref[...] = h


def _post_call(xji, agg, x, mats, vecs):
    n = x.shape[0]
    grid = (n // _BA,)
    row_spec = pl.BlockSpec((_BA, H), lambda i: (i, 0))
    fullm = pl.BlockSpec((H, H), lambda i: (0, 0))
    fullv = pl.BlockSpec((1, H), lambda i: (0, 0))
    bw1, bw2, lw, aw1a, aw2a, aw1b, aw2b = mats
    bb1, bb2, lb, ab1a, ab2a, ab1b, ab2b = vecs
    ops = [bw1, bb1, bw2, bb2, lw, lb, aw1a, ab1a, aw2a, ab2a, aw1b, ab1b, aw2b, ab2b]
    specs = [fullm, fullv, fullm, fullv, fullm, fullv,
             fullm, fullv, fullm, fullv, fullm, fullv, fullm, fullv]
    return pl.pallas_call(
        _post_body,
        grid=grid,
        in_specs=[row_spec, row_spec, row_spec] + specs,
        out_specs=row_spec,
        out_shape=jax.ShapeDtypeStruct((n, H), jnp.float32),
    )(xji, agg, x, *ops)


_GCHUNK = 128
_G_PER_W = N_TRIP // NW
_G_FULL = _G_PER_W // _GCHUNK
_G_TAIL = _G_PER_W - _G_FULL * _GCHUNK


def _sc_gather(table, idx):
    mesh = plsc.VectorSubcoreMesh(core_axis_name="c", subcore_axis_name="s")

    @functools.partial(
        pl.kernel, mesh=mesh,
        out_type=jax.ShapeDtypeStruct((N_TRIP, H), jnp.float32),
        scratch_types=[
            pltpu.VMEM((_GCHUNK,), jnp.int32),
            pltpu.VMEM((_GCHUNK, H), jnp.float32),
            pltpu.VMEM((_G_TAIL,), jnp.int32),
            pltpu.VMEM((_G_TAIL, H), jnp.float32),
            pltpu.SemaphoreType.DMA,
        ],
    )
    def gather_k(table_hbm, idx_hbm, out_hbm, idx_v, rows_v, idxt_v, rowst_v, sem):
        wid = lax.axis_index("s") * NC + lax.axis_index("c")
        base = wid * _G_PER_W

        def body(i, _):
            off = base + i * _GCHUNK
            pltpu.sync_copy(idx_hbm.at[pl.ds(off, _GCHUNK)], idx_v)
            pltpu.async_copy(table_hbm.at[idx_v], rows_v, sem).wait()
            pltpu.sync_copy(rows_v, out_hbm.at[pl.ds(off, _GCHUNK)])
            return 0

        lax.fori_loop(0, _G_FULL, body, 0)
        toff = base + _G_FULL * _GCHUNK
        pltpu.sync_copy(idx_hbm.at[pl.ds(toff, _G_TAIL)], idxt_v)
        pltpu.async_copy(table_hbm.at[idxt_v], rowst_v, sem).wait()
        pltpu.sync_copy(rowst_v, out_hbm.at[pl.ds(toff, _G_TAIL)])

    return gather_k(table, idx)


_R = 13952
_NPASS = 6
_E_PAD = NC * _NPASS * _R
_SCHUNK = 128
_S_PER_W = N_TRIP // NS
_S_FULL = _S_PER_W // _SCHUNK
_S_TAIL = _S_PER_W - _S_FULL * _SCHUNK
_ZROWS = (_R + NS) // NS


def _sc_scatter(msg, idx):
    mesh = plsc.VectorSubcoreMesh(core_axis_name="c", subcore_axis_name="s")
    zeros_blk = jnp.zeros((_ZROWS, H), jnp.float32)

    @functools.partial(
        pl.kernel, mesh=mesh,
        out_type=jax.ShapeDtypeStruct((_E_PAD, H), jnp.float32),
        scratch_types=[
            pltpu.VMEM((_SCHUNK,), jnp.int32),
            pltpu.VMEM((_SCHUNK,), jnp.int32),
            pltpu.VMEM((_SCHUNK, H), jnp.float32),
            pltpu.VMEM((_S_TAIL,), jnp.int32),
            pltpu.VMEM((_S_TAIL,), jnp.int32),
            pltpu.VMEM((_S_TAIL, H), jnp.float32),
            pltpu.VMEM_SHARED((_R + NS, H), jnp.float32),
        ],
    )
    def scatter_k(msg_hbm, idx_hbm, zeros_hbm, out_hbm,
                  idx_v, idx2_v, msg_v, idxt_v, idx2t_v, msgt_v, acc):
        c = lax.axis_index("c")
        s = lax.axis_index("s")
        tbase = s * _S_PER_W

        for p in range(_NPASS):
            base = (p * NC) * _R + c * _R

            pltpu.sync_copy(zeros_hbm, acc.at[pl.ds(s * _ZROWS, _ZROWS)])
            plsc.subcore_barrier()

            def chunk(i, _):
                coff = i * _SCHUNK
                pltpu.sync_copy(idx_hbm.at[pl.ds(tbase + coff, _SCHUNK)], idx_v)
                pltpu.sync_copy(msg_hbm.at[pl.ds(tbase + coff, _SCHUNK)], msg_v)
                for k in range(_SCHUNK // 16):
                    v = idx_v[pl.ds(k * 16, 16)]
                    rel = v - base
                    ok = (rel >= 0) & (rel < _R)
                    idx2_v[pl.ds(k * 16, 16)] = jnp.where(ok, rel, _R + s)
                pltpu.sync_copy(msg_v, acc.at[idx2_v], add=True)
                return 0

            lax.fori_loop(0, _S_FULL, chunk, 0)

            toff = _S_FULL * _SCHUNK
            pltpu.sync_copy(idx_hbm.at[pl.ds(tbase + toff, _S_TAIL)], idxt_v)
            pltpu.sync_copy(msg_hbm.at[pl.ds(tbase + toff, _S_TAIL)], msgt_v)
            v = idxt_v[...]
            rel = v - base
            ok = (rel >= 0) & (rel < _R)
            idx2t_v[...] = jnp.where(ok, rel, _R + s)
            pltpu.sync_copy(msgt_v, acc.at[idx2t_v], add=True)

            plsc.subcore_barrier()
            rows = _R // NS
            pltpu.sync_copy(acc.at[pl.ds(s * rows, rows)],
                            out_hbm.at[pl.ds(base + s * rows, rows)])
            plsc.subcore_barrier()

    return scatter_k(msg, idx, zeros_blk)[:N_EDGES]


def kernel(x, rbf, sbf, idx_kj, idx_ji, lin_rbf_w, lin_sbf_w, lin_ji_w,
           lin_ji_b, lin_kj_w, lin_kj_b, W, before_w1, before_b1, before_w2,
           before_b2, lin_w, lin_b, after_w1, after_b1, after_w2, after_b2):
    f32 = jnp.float32
    idx_kj = idx_kj.astype(jnp.int32)
    idx_ji = idx_ji.astype(jnp.int32)

    wjiT = lin_ji_w.T.astype(f32)
    wkjT = lin_kj_w.T.astype(f32)
    wrbfT = lin_rbf_w.T.astype(f32)
    wsbfT = lin_sbf_w.T.astype(f32)
    w2 = W.transpose(1, 2, 0).reshape(8 * H, H).astype(f32)

    bji = lin_ji_b.reshape(1, H)
    bkj = lin_kj_b.reshape(1, H)

    x_ji, x_kj = _pre_call(x, rbf, wjiT, bji, wkjT, bkj, wrbfT)
    gathered = _sc_gather(x_kj, idx_kj)
    msg = _einsum_call(gathered, sbf, wsbfT, w2)
    agg = _sc_scatter(msg, idx_ji)

    mats = (before_w1[0].T, before_w2[0].T, lin_w.T,
            after_w1[0].T, after_w2[0].T, after_w1[1].T, after_w2[1].T)
    vecs = (before_b1[0].reshape(1, H), before_b2[0].reshape(1, H),
            lin_b.reshape(1, H),
            after_b1[0].reshape(1, H), after_b2[0].reshape(1, H),
            after_b1[1].reshape(1, H), after_b2[1].reshape(1, H))
    return _post_call(x_ji, agg, x, mats, vecs)
